# baseline (speedup 1.0000x reference)
import functools

import jax
import jax.numpy as jnp
from jax import lax
from jax.experimental import pallas as pl
from jax.experimental.pallas import tpu as pltpu
from jax.experimental.pallas import tpu_sc as plsc

B = 4096
D_IN = 512
C = 1000
CPAD = 1024
EMB = 512
K = 10
KPAD = 16
NEG = -3.0e38

R = 128


def _tc_body(wt_ref, at_ref, b_ref, tags_ref, idx_ref):
    logits = jnp.dot(wt_ref[...], at_ref[...],
                     preferred_element_type=jnp.float32) + b_ref[...]
    m = jnp.max(logits, axis=0, keepdims=True)
    e = jnp.exp(logits - m)
    s = jnp.sum(e, axis=0, keepdims=True)
    tags_ref[...] = (e / s)[:C, :]

    row = lax.broadcasted_iota(jnp.int32, (CPAD, R), 0)
    work = logits
    picks = []
    for _ in range(K):
        mx = jnp.max(work, axis=0, keepdims=True)
        am = jnp.min(jnp.where(work == mx, row, CPAD), axis=0, keepdims=True)
        picks.append(am)
        work = jnp.where(row == am, NEG, work)
    picks.append(jnp.zeros((KPAD - K, R), jnp.int32))
    idx_ref[...] = jnp.concatenate(picks, axis=0)


def _tc_call(wt_pad, at, bt_pad):
    return pl.pallas_call(
        _tc_body,
        grid=(B // R,),
        in_specs=[
            pl.BlockSpec((CPAD, D_IN), lambda i: (0, 0)),
            pl.BlockSpec((D_IN, R), lambda i: (0, i)),
            pl.BlockSpec((CPAD, 1), lambda i: (0, 0)),
        ],
        out_specs=[
            pl.BlockSpec((C, R), lambda i: (0, i)),
            pl.BlockSpec((KPAD, R), lambda i: (0, i)),
        ],
        out_shape=[
            jax.ShapeDtypeStruct((C, B), jnp.float32),
            jax.ShapeDtypeStruct((KPAD, B), jnp.int32),
        ],
    )(wt_pad, at, bt_pad)


_NC = 2
_NS = 16
NW = _NC * _NS
BPW = B // NW
GCH = 32
NBUF = 4

_sc_mesh = plsc.VectorSubcoreMesh(core_axis_name="c", subcore_axis_name="s")


@functools.partial(
    pl.kernel,
    mesh=_sc_mesh,
    out_type=jax.ShapeDtypeStruct((K, B, EMB), jnp.float32),
    scratch_types=[pltpu.VMEM((KPAD, BPW), jnp.int32)]
    + [pltpu.VMEM((GCH, EMB), jnp.float32)] * NBUF
    + [pltpu.SemaphoreType.DMA] * (2 * NBUF),
)
def _sc_gather(table_hbm, idx_hbm, out_hbm, idx_v, *bufs):
    rows = bufs[:NBUF]
    gsem = bufs[NBUF:2 * NBUF]
    osem = bufs[2 * NBUF:]
    wid = lax.axis_index("s") * _NC + lax.axis_index("c")
    base = wid * BPW
    pltpu.sync_copy(idx_hbm.at[pl.ds(0, KPAD), pl.ds(base, BPW)], idx_v)

    nch = K * (BPW // GCH)

    @pl.loop(0, nch, step=NBUF)
    def _chunk(j):
        for t in range(NBUF):
            u = j + t
            k = u // (BPW // GCH)
            half = u % (BPW // GCH)

            @pl.when(j > 0)
            def _():
                up = u - NBUF
                pltpu.make_async_copy(
                    rows[t],
                    out_hbm.at[up // (BPW // GCH),
                               pl.ds(base + (up % (BPW // GCH)) * GCH, GCH)],
                    osem[t]).wait()

            pltpu.async_copy(
                table_hbm.at[idx_v.at[k, pl.ds(half * GCH, GCH)]],
                rows[t], gsem[t])
        for t in range(NBUF):
            u = j + t
            k = u // (BPW // GCH)
            half = u % (BPW // GCH)
            pltpu.make_async_copy(
                table_hbm.at[idx_v.at[0, pl.ds(0, GCH)]],
                rows[t], gsem[t]).wait()
            pltpu.async_copy(
                rows[t], out_hbm.at[k, pl.ds(base + half * GCH, GCH)],
                osem[t])

    for t in range(NBUF):
        ul = nch - NBUF + t
        pltpu.make_async_copy(
            rows[t],
            out_hbm.at[ul // (BPW // GCH),
                       pl.ds(base + (ul % (BPW // GCH)) * GCH, GCH)],
            osem[t]).wait()


def kernel(avg_features, W, b, embed_table):
    wt_pad = jnp.pad(W.T, ((0, CPAD - C), (0, 0)))
    bt_pad = jnp.pad(b, (0, CPAD - C), constant_values=NEG).reshape(CPAD, 1)
    tags_t, idx_t = _tc_call(wt_pad, avg_features.T, bt_pad)
    out_km = _sc_gather(embed_table, idx_t)
    return tags_t.T, out_km.transpose(1, 0, 2)


# speedup vs baseline: 1.1222x; 1.1222x over previous
import functools

import jax
import jax.numpy as jnp
from jax import lax
from jax.experimental import pallas as pl
from jax.experimental.pallas import tpu as pltpu
from jax.experimental.pallas import tpu_sc as plsc

B = 4096
D_IN = 512
C = 1000
CPAD = 1024
EMB = 512
K = 10
KPAD = 16
NEG = -3.0e38

R = 256


def _tc_topk_body(wt_ref, at_ref, b_ref, idx_ref):
    logits = jnp.dot(wt_ref[...], at_ref[...],
                     preferred_element_type=jnp.float32) + b_ref[...]
    row = lax.broadcasted_iota(jnp.int32, (CPAD, R), 0)
    work = logits
    picks = []
    for _ in range(K):
        mx = jnp.max(work, axis=0, keepdims=True)
        am = jnp.min(jnp.where(work == mx, row, CPAD), axis=0, keepdims=True)
        picks.append(am)
        work = jnp.where(row == am, NEG, work)
    picks.append(jnp.zeros((KPAD - K, R), jnp.int32))
    idx_ref[...] = jnp.concatenate(picks, axis=0)


def _tc_tags_body(wt_ref, at_ref, b_ref, tags_ref):
    logits = jnp.dot(wt_ref[...], at_ref[...],
                     preferred_element_type=jnp.float32) + b_ref[...]
    m = jnp.max(logits, axis=0, keepdims=True)
    e = jnp.exp(logits - m)
    s = jnp.sum(e, axis=0, keepdims=True)
    tags_ref[...] = (e / s)[:C, :]


_TC_IN_SPECS = [
    pl.BlockSpec((CPAD, D_IN), lambda i: (0, 0)),
    pl.BlockSpec((D_IN, R), lambda i: (0, i)),
    pl.BlockSpec((CPAD, 1), lambda i: (0, 0)),
]


def _tc_topk(wt_pad, at, bt_pad):
    return pl.pallas_call(
        _tc_topk_body,
        grid=(B // R,),
        in_specs=_TC_IN_SPECS,
        out_specs=pl.BlockSpec((KPAD, R), lambda i: (0, i)),
        out_shape=jax.ShapeDtypeStruct((KPAD, B), jnp.int32),
    )(wt_pad, at, bt_pad)


def _tc_tags(wt_pad, at, bt_pad):
    return pl.pallas_call(
        _tc_tags_body,
        grid=(B // R,),
        in_specs=_TC_IN_SPECS,
        out_specs=pl.BlockSpec((C, R), lambda i: (0, i)),
        out_shape=jax.ShapeDtypeStruct((C, B), jnp.float32),
    )(wt_pad, at, bt_pad)


_NC = 2
_NS = 16
NW = _NC * _NS
BPW = B // NW
GCH = 32
NBUF = 4

_sc_mesh = plsc.VectorSubcoreMesh(core_axis_name="c", subcore_axis_name="s")


@functools.partial(
    pl.kernel,
    mesh=_sc_mesh,
    out_type=jax.ShapeDtypeStruct((K, B, EMB), jnp.float32),
    scratch_types=[pltpu.VMEM((KPAD, BPW), jnp.int32)]
    + [pltpu.VMEM((GCH, EMB), jnp.float32)] * NBUF
    + [pltpu.SemaphoreType.DMA] * (2 * NBUF),
)
def _sc_gather(table_hbm, idx_hbm, out_hbm, idx_v, *bufs):
    rows = bufs[:NBUF]
    gsem = bufs[NBUF:2 * NBUF]
    osem = bufs[2 * NBUF:]
    wid = lax.axis_index("s") * _NC + lax.axis_index("c")
    base = wid * BPW
    pltpu.sync_copy(idx_hbm.at[pl.ds(0, KPAD), pl.ds(base, BPW)], idx_v)

    nch = K * (BPW // GCH)

    @pl.loop(0, nch, step=NBUF)
    def _chunk(j):
        for t in range(NBUF):
            u = j + t
            k = u // (BPW // GCH)
            half = u % (BPW // GCH)

            @pl.when(j > 0)
            def _():
                up = u - NBUF
                pltpu.make_async_copy(
                    rows[t],
                    out_hbm.at[up // (BPW // GCH),
                               pl.ds(base + (up % (BPW // GCH)) * GCH, GCH)],
                    osem[t]).wait()

            pltpu.async_copy(
                table_hbm.at[idx_v.at[k, pl.ds(half * GCH, GCH)]],
                rows[t], gsem[t])
        for t in range(NBUF):
            u = j + t
            k = u // (BPW // GCH)
            half = u % (BPW // GCH)
            pltpu.make_async_copy(
                table_hbm.at[idx_v.at[0, pl.ds(0, GCH)]],
                rows[t], gsem[t]).wait()
            pltpu.async_copy(
                rows[t], out_hbm.at[k, pl.ds(base + half * GCH, GCH)],
                osem[t])

    for t in range(NBUF):
        ul = nch - NBUF + t
        pltpu.make_async_copy(
            rows[t],
            out_hbm.at[ul // (BPW // GCH),
                       pl.ds(base + (ul % (BPW // GCH)) * GCH, GCH)],
            osem[t]).wait()


def kernel(avg_features, W, b, embed_table):
    wt_pad = jnp.pad(W.T, ((0, CPAD - C), (0, 0)))
    bt_pad = jnp.pad(b, (0, CPAD - C), constant_values=NEG).reshape(CPAD, 1)
    at = avg_features.T
    idx_t = _tc_topk(wt_pad, at, bt_pad)
    out_km = _sc_gather(embed_table, idx_t)
    tags_t = _tc_tags(wt_pad, at, bt_pad)
    return tags_t.T, out_km.transpose(1, 0, 2)


# speedup vs baseline: 1.1489x; 1.0238x over previous
import functools

import jax
import jax.numpy as jnp
from jax import lax
from jax.experimental import pallas as pl
from jax.experimental.pallas import tpu as pltpu
from jax.experimental.pallas import tpu_sc as plsc

B = 4096
D_IN = 512
C = 1000
CPAD = 1024
EMB = 512
K = 10
KPAD = 16
NEG = -3.0e38

R = 256


def _tc_body(wt_ref, at_ref, b_ref, tags_ref, idx_ref):
    logits = jnp.dot(wt_ref[...], at_ref[...],
                     preferred_element_type=jnp.float32) + b_ref[...]
    m = jnp.max(logits, axis=0, keepdims=True)
    e = jnp.exp(logits - m)
    s = jnp.sum(e, axis=0, keepdims=True)
    tags_ref[...] = (e / s)[:C, :]

    row = lax.broadcasted_iota(jnp.int32, (CPAD, R), 0)
    work = logits
    picks = []
    for _ in range(K):
        mx = jnp.max(work, axis=0, keepdims=True)
        am = jnp.min(jnp.where(work == mx, row, CPAD), axis=0, keepdims=True)
        picks.append(am)
        work = jnp.where(row == am, NEG, work)
    picks.append(jnp.zeros((KPAD - K, R), jnp.int32))
    idx_ref[...] = jnp.concatenate(picks, axis=0)


def _tc_call(wt_pad, at, bt_pad):
    return pl.pallas_call(
        _tc_body,
        grid=(B // R,),
        in_specs=[
            pl.BlockSpec((CPAD, D_IN), lambda i: (0, 0)),
            pl.BlockSpec((D_IN, R), lambda i: (0, i)),
            pl.BlockSpec((CPAD, 1), lambda i: (0, 0)),
        ],
        out_specs=[
            pl.BlockSpec((C, R), lambda i: (0, i)),
            pl.BlockSpec((KPAD, R), lambda i: (0, i)),
        ],
        out_shape=[
            jax.ShapeDtypeStruct((C, B), jnp.float32),
            jax.ShapeDtypeStruct((KPAD, B), jnp.int32),
        ],
    )(wt_pad, at, bt_pad)


_NC = 2
_NS = 16
NW = _NC * _NS
BPW = B // NW
GCH = 32
NBUF = 4

_sc_mesh = plsc.VectorSubcoreMesh(core_axis_name="c", subcore_axis_name="s")


@functools.partial(
    pl.kernel,
    mesh=_sc_mesh,
    out_type=jax.ShapeDtypeStruct((K, B, EMB), jnp.float32),
    scratch_types=[pltpu.VMEM((KPAD, BPW), jnp.int32)]
    + [pltpu.VMEM((GCH, EMB), jnp.float32)] * NBUF
    + [pltpu.SemaphoreType.DMA] * (2 * NBUF),
)
def _sc_gather(table_hbm, idx_hbm, out_hbm, idx_v, *bufs):
    rows = bufs[:NBUF]
    gsem = bufs[NBUF:2 * NBUF]
    osem = bufs[2 * NBUF:]
    wid = lax.axis_index("s") * _NC + lax.axis_index("c")
    base = wid * BPW
    pltpu.sync_copy(idx_hbm.at[pl.ds(0, KPAD), pl.ds(base, BPW)], idx_v)

    nch = K * (BPW // GCH)

    @pl.loop(0, nch, step=NBUF)
    def _chunk(j):
        for t in range(NBUF):
            u = j + t
            k = u // (BPW // GCH)
            half = u % (BPW // GCH)

            @pl.when(j > 0)
            def _():
                up = u - NBUF
                pltpu.make_async_copy(
                    rows[t],
                    out_hbm.at[up // (BPW // GCH),
                               pl.ds(base + (up % (BPW // GCH)) * GCH, GCH)],
                    osem[t]).wait()

            pltpu.async_copy(
                table_hbm.at[idx_v.at[k, pl.ds(half * GCH, GCH)]],
                rows[t], gsem[t])
        for t in range(NBUF):
            u = j + t
            k = u // (BPW // GCH)
            half = u % (BPW // GCH)
            pltpu.make_async_copy(
                table_hbm.at[idx_v.at[0, pl.ds(0, GCH)]],
                rows[t], gsem[t]).wait()
            pltpu.async_copy(
                rows[t], out_hbm.at[k, pl.ds(base + half * GCH, GCH)],
                osem[t])

    for t in range(NBUF):
        ul = nch - NBUF + t
        pltpu.make_async_copy(
            rows[t],
            out_hbm.at[ul // (BPW // GCH),
                       pl.ds(base + (ul % (BPW // GCH)) * GCH, GCH)],
            osem[t]).wait()


def kernel(avg_features, W, b, embed_table):
    wt_pad = jnp.pad(W.T, ((0, CPAD - C), (0, 0)))
    bt_pad = jnp.pad(b, (0, CPAD - C), constant_values=NEG).reshape(CPAD, 1)
    tags_t, idx_t = _tc_call(wt_pad, avg_features.T, bt_pad)
    out_km = _sc_gather(embed_table, idx_t)
    return tags_t.T, out_km.transpose(1, 0, 2)
